# baseline (speedup 1.0000x reference)
import functools
import math

import jax
import jax.numpy as jnp
import numpy as np
from jax import lax
from jax.experimental import pallas as pl
from jax.experimental.pallas import tpu as pltpu
from jax.experimental.pallas import tpu_sc as plsc

B = 4096
L = 50
D = 128
NC = 2
NS = 16
NW = NC * NS
EPW = B // NW
LP = 64
LS = 56
SCALE = float(np.sqrt(np.float32(D)))
NEG = -1e30


def _pos_encoding():
    pos = np.arange(200)[:, np.newaxis]
    i = np.arange(D)[np.newaxis, :]
    angle_rates = 1.0 / np.power(10000, 2 * (i // 2) / np.float32(D))
    ang = pos * angle_rates
    ang[:, 0::2] = np.sin(ang[:, 0::2])
    ang[:, 1::2] = np.cos(ang[:, 1::2])
    return ang[:L].astype(np.float32)


_PE = _pos_encoding()
_PE_PAD = np.zeros((LP, D), np.float32)
_PE_PAD[:L] = _PE


def _sc_walayer(x32, table, wvec, peb):
    mesh = plsc.VectorSubcoreMesh(core_axis_name="c", subcore_axis_name="s")

    @functools.partial(
        pl.kernel,
        mesh=mesh,
        out_type=[
            jax.ShapeDtypeStruct((NW, EPW * LP), jnp.float32),
            jax.ShapeDtypeStruct((NW, EPW * D), jnp.float32),
        ],
        scratch_types=[
            pltpu.VMEM((EPW * LS,), jnp.int32),
            pltpu.VMEM((4, L, D), jnp.float32),
            pltpu.VMEM((D,), jnp.float32),
            pltpu.VMEM((LP,), jnp.float32),
            pltpu.VMEM((EPW * LP + 16,), jnp.float32),
            pltpu.VMEM((EPW * D,), jnp.float32),
            pltpu.SemaphoreType.DMA,
            pltpu.SemaphoreType.DMA,
            pltpu.SemaphoreType.DMA,
            pltpu.SemaphoreType.DMA,
        ],
    )
    def k(x_hbm, tab_hbm, w_hbm, peb_hbm, attn_hbm, wsum_hbm,
          idx_v, rows2_v, w_v, peb_v, attn_v, wsum_v,
          sem0, sem1, sem2, sem3):
        wid = lax.axis_index("s") * NC + lax.axis_index("c")
        pltpu.sync_copy(x_hbm.at[wid], idx_v)
        pltpu.sync_copy(w_hbm, w_v)
        pltpu.sync_copy(peb_hbm, peb_v)
        lio = lax.broadcasted_iota(jnp.int32, (16,), 0)
        negv = jnp.full((16,), NEG, jnp.float32)

        dnums = lax.GatherDimensionNumbers(
            offset_dims=(), collapsed_slice_dims=(0,), start_index_map=(0,))

        def perm(v, idx):
            return lax.gather(
                v, idx[:, None], dnums, (1,),
                mode=lax.GatherScatterMode.PROMISE_IN_BOUNDS)

        def lane_reduce(v, op):
            for k in (1, 2, 4, 8):
                v = op(v, perm(v, lio ^ k))
            return v

        def start(e, buf, sem):
            pltpu.async_copy(
                tab_hbm.at[idx_v.at[pl.ds(e * LS, L)]], rows2_v.at[buf], sem)

        def wait(buf, sem):
            pltpu.make_async_copy(
                tab_hbm.at[idx_v.at[pl.ds(0, L)]], rows2_v.at[buf], sem
            ).wait()

        def tree_lane_sums(accs):
            vecs = accs
            k = 1
            while len(vecs) > 1:
                nxt = []
                for q in range(0, len(vecs), 2):
                    a, b = vecs[q], vecs[q + 1]
                    msk = (lio & k) == 0
                    nxt.append(jnp.where(msk, a + perm(a, lio ^ k),
                                         b + perm(b, lio ^ k)))
                vecs = nxt
                k *= 2
            v = vecs[0]
            while k < 16:
                v = v + perm(v, lio ^ k)
                k *= 2
            return v

        def compute(e, rows_v):
            wc = [w_v[pl.ds(c * 16, 16)] for c in range(8)]

            def dot_row(l):
                p = [rows_v[l, pl.ds(c * 16, 16)] * wc[c] for c in range(8)]
                while len(p) > 1:
                    p = [p[i] + p[i + 1] for i in range(0, len(p), 2)]
                return p[0]

            lgt = []
            for g in range(4):
                vec = negv
                for j in range(16):
                    l = g * 16 + j
                    if l >= L:
                        break
                    vec = jnp.where(lio == j,
                                    lane_reduce(dot_row(l), jnp.add), vec)
                lgt.append(vec)
            pebg = [peb_v[pl.ds(g * 16, 16)] for g in range(4)]
            for g in range(3):
                lgt[g] = lgt[g] + pebg[g]
            lgt[3] = jnp.where(lio < 2, lgt[3] + pebg[3], lgt[3])

            mv = lane_reduce(jnp.maximum(jnp.maximum(lgt[0], lgt[1]),
                                         jnp.maximum(lgt[2], lgt[3])),
                             jnp.maximum)
            ev = [jnp.exp(v - mv) for v in lgt]
            sv = lane_reduce(ev[0] + ev[1] + ev[2] + ev[3], jnp.add)
            av = [e_ / sv for e_ in ev]
            for g in range(4):
                attn_v[pl.ds(e * LP + g * 16, 16)] = av[g]

            accs = [jnp.zeros((16,), jnp.float32) for _ in range(8)]
            for l in range(L):
                a = av[l // 16][l % 16]
                for c in range(8):
                    accs[c] = accs[c] + rows_v[l, pl.ds(c * 16, 16)] * a
            for c in range(8):
                wsum_v[pl.ds(e * D + c * 16, 16)] = accs[c]

        sems = (sem0, sem1, sem2, sem3)
        for b in range(4):
            start(b, b, sems[b])

        def pipe_body(i, _):
            e = i * 4
            for b in range(4):
                wait(b, sems[b])
                compute(e + b, rows2_v.at[b])

                @pl.when(e + b + 4 < EPW)
                def _():
                    start(e + b + 4, b, sems[b])

            return 0

        lax.fori_loop(0, EPW // 4, pipe_body, 0)
        pltpu.sync_copy(attn_v.at[pl.ds(0, EPW * LP)], attn_hbm.at[wid])
        pltpu.sync_copy(wsum_v, wsum_hbm.at[wid])

    return k(x32, table, wvec, peb)


def _tc_finish(wsum, attn64):
    BLK = 512

    def body(w_ref, a_ref, p_ref, o_ref):
        o_ref[...] = w_ref[...] * SCALE + jnp.dot(
            a_ref[...], p_ref[...], preferred_element_type=jnp.float32)

    return pl.pallas_call(
        body,
        grid=(B // BLK,),
        in_specs=[
            pl.BlockSpec((BLK, D), lambda i: (i, 0)),
            pl.BlockSpec((BLK, LP), lambda i: (i, 0)),
            pl.BlockSpec((LP, D), lambda i: (0, 0)),
        ],
        out_specs=pl.BlockSpec((BLK, D), lambda i: (i, 0)),
        out_shape=jax.ShapeDtypeStruct((B, D), jnp.float32),
    )(wsum, attn64, jnp.asarray(_PE_PAD))


def kernel(x, embedding_table, ff_w, ff_b):
    x32 = jnp.pad(x.astype(jnp.int32), ((0, 0), (0, LS - L))).reshape(NW, EPW * LS)
    w = ff_w[:, 0] * SCALE
    peb = jnp.pad((jnp.asarray(_PE) @ ff_w)[:, 0] + ff_b[0], (0, LP - L))
    attn_r, wsum_r = _sc_walayer(x32, embedding_table, w, peb)
    attn64 = attn_r.reshape(B, LP)
    wsum = wsum_r.reshape(B, D)
    out = _tc_finish(wsum, attn64)
    return out, attn64[:, :L]


# speedup vs baseline: 1.1296x; 1.1296x over previous
import functools
import math

import jax
import jax.numpy as jnp
import numpy as np
from jax import lax
from jax.experimental import pallas as pl
from jax.experimental.pallas import tpu as pltpu
from jax.experimental.pallas import tpu_sc as plsc

B = 4096
L = 50
D = 128
NC = 2
NS = 16
NW = NC * NS
EPW = B // NW
LP = 64
LS = 56
SCALE = float(np.sqrt(np.float32(D)))
NEG = -1e30


def _pos_encoding():
    pos = np.arange(200)[:, np.newaxis]
    i = np.arange(D)[np.newaxis, :]
    angle_rates = 1.0 / np.power(10000, 2 * (i // 2) / np.float32(D))
    ang = pos * angle_rates
    ang[:, 0::2] = np.sin(ang[:, 0::2])
    ang[:, 1::2] = np.cos(ang[:, 1::2])
    return ang[:L].astype(np.float32)


_PE = _pos_encoding()
_PE_PAD = np.zeros((LP, D), np.float32)
_PE_PAD[:L] = _PE


def _sc_walayer(x32, table, wvec, peb):
    mesh = plsc.VectorSubcoreMesh(core_axis_name="c", subcore_axis_name="s")

    @functools.partial(
        pl.kernel,
        mesh=mesh,
        out_type=[
            jax.ShapeDtypeStruct((NW, EPW * LP), jnp.float32),
            jax.ShapeDtypeStruct((NW, EPW * D), jnp.float32),
        ],
        scratch_types=[
            pltpu.VMEM((EPW * LS,), jnp.int32),
            pltpu.VMEM((2, L, D), jnp.float32),
            pltpu.VMEM((D,), jnp.float32),
            pltpu.VMEM((LP,), jnp.float32),
            pltpu.VMEM((EPW * LP + 16,), jnp.float32),
            pltpu.VMEM((EPW * D,), jnp.float32),
            pltpu.SemaphoreType.DMA,
            pltpu.SemaphoreType.DMA,
        ],
    )
    def k(x_hbm, tab_hbm, w_hbm, peb_hbm, attn_hbm, wsum_hbm,
          idx_v, rows2_v, w_v, peb_v, attn_v, wsum_v, sem0, sem1):
        wid = lax.axis_index("s") * NC + lax.axis_index("c")
        pltpu.sync_copy(x_hbm.at[wid], idx_v)
        pltpu.sync_copy(w_hbm, w_v)
        pltpu.sync_copy(peb_hbm, peb_v)
        lio = lax.broadcasted_iota(jnp.int32, (16,), 0)
        negv = jnp.full((16,), NEG, jnp.float32)

        dnums = lax.GatherDimensionNumbers(
            offset_dims=(), collapsed_slice_dims=(0,), start_index_map=(0,))

        def perm(v, idx):
            return lax.gather(
                v, idx[:, None], dnums, (1,),
                mode=lax.GatherScatterMode.PROMISE_IN_BOUNDS)

        def lane_reduce(v, op):
            for k in (1, 2, 4, 8):
                v = op(v, perm(v, lio ^ k))
            return v

        def start(e, buf, sem):
            pltpu.async_copy(
                tab_hbm.at[idx_v.at[pl.ds(e * LS, L)]], rows2_v.at[buf], sem)

        def wait(buf, sem):
            pltpu.make_async_copy(
                tab_hbm.at[idx_v.at[pl.ds(0, L)]], rows2_v.at[buf], sem
            ).wait()

        def compute(e, rows_v):
            wc = [w_v[pl.ds(c * 16, 16)] for c in range(8)]

            def dot_row(l):
                p = [rows_v[l, pl.ds(c * 16, 16)] * wc[c] for c in range(8)]
                while len(p) > 1:
                    p = [p[i] + p[i + 1] for i in range(0, len(p), 2)]
                return p[0]

            lgt = []
            for g in range(4):
                vec = negv
                for j in range(16):
                    l = g * 16 + j
                    if l >= L:
                        break
                    vec = jnp.where(lio == j,
                                    lane_reduce(dot_row(l), jnp.add), vec)
                lgt.append(vec)
            pebg = [peb_v[pl.ds(g * 16, 16)] for g in range(4)]
            for g in range(3):
                lgt[g] = lgt[g] + pebg[g]
            lgt[3] = jnp.where(lio < 2, lgt[3] + pebg[3], lgt[3])

            mv = lane_reduce(jnp.maximum(jnp.maximum(lgt[0], lgt[1]),
                                         jnp.maximum(lgt[2], lgt[3])),
                             jnp.maximum)
            ev = [jnp.exp(v - mv) for v in lgt]
            sv = lane_reduce(ev[0] + ev[1] + ev[2] + ev[3], jnp.add)
            av = [e_ / sv for e_ in ev]
            for g in range(4):
                attn_v[pl.ds(e * LP + g * 16, 16)] = av[g]

            accs = [jnp.zeros((16,), jnp.float32) for _ in range(8)]
            for l in range(L):
                a = av[l // 16][l % 16]
                for c in range(8):
                    accs[c] = accs[c] + rows_v[l, pl.ds(c * 16, 16)] * a
            for c in range(8):
                wsum_v[pl.ds(e * D + c * 16, 16)] = accs[c]

        sems = (sem0, sem1)
        for b in range(2):
            start(b, b, sems[b])

        def pipe_body(i, _):
            e = i * 2
            for b in range(2):
                wait(b, sems[b])
                compute(e + b, rows2_v.at[b])

                @pl.when(e + b + 2 < EPW)
                def _():
                    start(e + b + 2, b, sems[b])

            return 0

        lax.fori_loop(0, EPW // 2, pipe_body, 0)
        pltpu.sync_copy(attn_v.at[pl.ds(0, EPW * LP)], attn_hbm.at[wid])
        pltpu.sync_copy(wsum_v, wsum_hbm.at[wid])

    return k(x32, table, wvec, peb)


def _tc_finish(wsum, attn64):
    BLK = 512

    def body(w_ref, a_ref, p_ref, o_ref):
        o_ref[...] = w_ref[...] * SCALE + jnp.dot(
            a_ref[...], p_ref[...], preferred_element_type=jnp.float32)

    return pl.pallas_call(
        body,
        grid=(B // BLK,),
        in_specs=[
            pl.BlockSpec((BLK, D), lambda i: (i, 0)),
            pl.BlockSpec((BLK, LP), lambda i: (i, 0)),
            pl.BlockSpec((LP, D), lambda i: (0, 0)),
        ],
        out_specs=pl.BlockSpec((BLK, D), lambda i: (i, 0)),
        out_shape=jax.ShapeDtypeStruct((B, D), jnp.float32),
    )(wsum, attn64, jnp.asarray(_PE_PAD))


def kernel(x, embedding_table, ff_w, ff_b):
    x32 = jnp.pad(x.astype(jnp.int32), ((0, 0), (0, LS - L))).reshape(NW, EPW * LS)
    w = ff_w[:, 0] * SCALE
    peb = jnp.pad((jnp.asarray(_PE) @ ff_w)[:, 0] + ff_b[0], (0, LP - L))
    attn_r, wsum_r = _sc_walayer(x32, embedding_table, w, peb)
    attn64 = attn_r.reshape(B, LP)
    wsum = wsum_r.reshape(B, D)
    out = _tc_finish(wsum, attn64)
    return out, attn64[:, :L]
